# Initial kernel scaffold; baseline (speedup 1.0000x reference)
import jax
import jax.numpy as jnp
from jax.experimental import pallas as pl


def kernel(edge_ptr, src_edges, dst_nodes, input_feat, weight, neighbor_num):
    raise NotImplementedError("write your pallas kernel here")


# speedup vs baseline: 3.0273x; 3.0273x over previous
import functools

import jax
import jax.numpy as jnp
from jax import lax
from jax.experimental import pallas as pl
from jax.experimental.pallas import tpu as pltpu
from jax.experimental.pallas import tpu_sc as plsc

N = 10000
E = 160000
D = 256
DO = 256

NC = 2
NS = 16
NPW = 320
NPC = NPW * NS
N_PAD = NPC * NC
C = 128
PTR_LEN = NPW + 16
PTR_PAD = N_PAD + 16
SRC_PAD = E + C


def _sc_agg_body(ptr_hbm, src_hbm, feat_hbm, agg_hbm,
                 ptr_v, ptr_s, idx_v, rows_v, acc_v, sem):
    c = lax.axis_index("c")
    s = lax.axis_index("s")
    nb = c * NPC + s * NPW

    pltpu.sync_copy(ptr_hbm.at[pl.ds(nb, PTR_LEN)], ptr_v)
    iota = lax.iota(jnp.int32, 16)
    for j in range(PTR_LEN // 16):
        v = ptr_v[pl.ds(j * 16, 16)]
        for l in range(16):
            ptr_s[j * 16 + l] = jnp.sum(jnp.where(iota == l, v, 0))

    def zrow(i, zero):
        for k in range(D // 16):
            acc_v[i, pl.ds(k * 16, 16)] = jnp.zeros((16,), jnp.float32)
        return zero

    lax.fori_loop(0, NPW, zrow, jnp.int32(0))

    e0 = ptr_s[0]
    e1 = ptr_s[NPW]
    a0 = (e0 // 8) * 8
    nch = (e1 - a0 + (C - 1)) // C

    def chunk(g, cur):
        base = a0 + g * C
        pltpu.sync_copy(src_hbm.at[pl.ds(base, C)], idx_v)
        pltpu.async_copy(feat_hbm.at[idx_v], rows_v, sem).wait()

        def edge(p, cc):
            gpos = base + p
            cc = lax.while_loop(
                lambda n: (ptr_s[n + 1] <= gpos) & (n + 1 < NPW),
                lambda n: n + 1, cc)

            @pl.when((gpos >= e0) & (gpos < e1))
            def _():
                for k in range(D // 16):
                    sl = pl.ds(k * 16, 16)
                    acc_v[cc, sl] = acc_v[cc, sl] + rows_v[p, sl]

            return cc

        return lax.fori_loop(0, C, edge, cur)

    lax.fori_loop(0, nch, chunk, jnp.int32(0))

    pltpu.sync_copy(acc_v, agg_hbm.at[pl.ds(nb, NPW)])


@functools.cache
def _make_sc_agg():
    return pl.kernel(
        _sc_agg_body,
        out_type=jax.ShapeDtypeStruct((N_PAD, D), jnp.float32),
        mesh=plsc.VectorSubcoreMesh(core_axis_name="c", subcore_axis_name="s",
                                    num_cores=NC, num_subcores=NS),
        compiler_params=pltpu.CompilerParams(needs_layout_passes=False),
        scratch_types=[
            pltpu.VMEM((PTR_LEN,), jnp.int32),
            pltpu.SMEM((PTR_LEN,), jnp.int32),
            pltpu.VMEM((C,), jnp.int32),
            pltpu.VMEM((C, D), jnp.float32),
            pltpu.VMEM((NPW, D), jnp.float32),
            pltpu.SemaphoreType.DMA,
        ],
    )


BM = 2000


def _mm_body(x_ref, agg_ref, w_ref, o_ref):
    o_ref[...] = jnp.dot(x_ref[...] + agg_ref[...], w_ref[...],
                         preferred_element_type=jnp.float32)


def _matmul(x, agg, w):
    return pl.pallas_call(
        _mm_body,
        grid=(N // BM,),
        in_specs=[
            pl.BlockSpec((BM, D), lambda i: (i, 0)),
            pl.BlockSpec((BM, D), lambda i: (i, 0)),
            pl.BlockSpec((D, DO), lambda i: (0, 0)),
        ],
        out_specs=pl.BlockSpec((BM, DO), lambda i: (i, 0)),
        out_shape=jax.ShapeDtypeStruct((N, DO), jnp.float32),
    )(x, agg, w)


@jax.jit
def kernel(edge_ptr, src_edges, dst_nodes, input_feat, weight, neighbor_num):
    del dst_nodes, neighbor_num
    ptr_pad = jnp.pad(edge_ptr.astype(jnp.int32), (0, PTR_PAD - (N + 1)),
                      constant_values=E)
    src_pad = jnp.pad(src_edges.astype(jnp.int32), (0, SRC_PAD - E))
    agg = _make_sc_agg()(ptr_pad, src_pad, input_feat)
    return _matmul(input_feat, agg, weight)


# speedup vs baseline: 6.1248x; 2.0232x over previous
import functools

import jax
import jax.numpy as jnp
from jax import lax
from jax.experimental import pallas as pl
from jax.experimental.pallas import tpu as pltpu
from jax.experimental.pallas import tpu_sc as plsc

N = 10000
E = 160000
D = 256
DO = 256
NG = D // 16

NC = 2
NS = 16
NPW = 320
NPC = NPW * NS
N_PAD = NPC * NC
C = 64
PTR_LEN = NPW + 16
PTR_PAD = N_PAD + 16
SRC_PAD = E + 2 * C
ZROW = N
ACC_ROWS = NPW + 8
FEAT_PAD = N + 8


def _sc_agg_body(ptr_hbm, src_hbm, feat_hbm, agg_hbm,
                 ptr_v, ptr_s, idx2, rows2, acc_v, sem0, sem1):
    c = lax.axis_index("c")
    s = lax.axis_index("s")
    nb = c * NPC + s * NPW

    pltpu.sync_copy(ptr_hbm.at[pl.ds(nb, PTR_LEN)], ptr_v)
    iota = lax.iota(jnp.int32, 16)
    for j in range(PTR_LEN // 16):
        v = ptr_v[pl.ds(j * 16, 16)]
        for l in range(16):
            ptr_s[j * 16 + l] = jnp.sum(jnp.where(iota == l, v, 0))

    def zrow(i, zero):
        for k in range(NG):
            acc_v[i, pl.ds(k * 16, 16)] = jnp.zeros((16,), jnp.float32)
        return zero

    lax.fori_loop(0, ACC_ROWS, zrow, jnp.int32(0))

    e0 = ptr_s[0]
    e1 = ptr_s[NPW]
    a0 = (e0 // 8) * 8
    npair = (e1 - a0 + (2 * C - 1)) // (2 * C)

    def prefetch(gc, off, sem):
        bb = a0 + gc * C
        pltpu.sync_copy(src_hbm.at[pl.ds(bb, C)], idx2.at[pl.ds(off, C)])
        for k in range(C // 16):
            sl = pl.ds(off + k * 16, 16)
            gpos = bb + k * 16 + iota
            ok = (gpos >= e0) & (gpos < e1)
            idx2[sl] = jnp.where(ok, idx2[sl], ZROW)
        pltpu.async_copy(feat_hbm.at[idx2.at[pl.ds(off, C)]],
                         rows2.at[pl.ds(off, C)], sem)

    def process(gc, off, sem, carry):
        pltpu.make_async_copy(feat_hbm.at[pl.ds(0, C)],
                              rows2.at[pl.ds(off, C)], sem).wait()
        bb = a0 + gc * C

        def edge(p, car):
            cur = car[0]
            acc = car[1:]
            gpos = bb + p
            close = ptr_s[cur + 1] <= gpos

            @pl.when(close)
            def _():
                for k in range(NG):
                    acc_v[cur, pl.ds(k * 16, 16)] = acc[k]

            cur = lax.while_loop(
                lambda n: (ptr_s[n + 1] <= gpos) & (n < NPW),
                lambda n: n + 1, cur)
            rp = off + p
            new_acc = tuple(
                jnp.where(close, 0.0, acc[k]) + rows2[rp, pl.ds(k * 16, 16)]
                for k in range(NG))
            return (cur,) + new_acc

        return lax.fori_loop(0, C, edge, carry)

    @pl.when(npair > 0)
    def _():
        prefetch(0, 0, sem0)

    def pair(q, carry):
        prefetch(2 * q + 1, C, sem1)
        carry = process(2 * q, 0, sem0, carry)

        @pl.when(2 * q + 2 < 2 * npair)
        def _():
            prefetch(2 * q + 2, 0, sem0)

        return process(2 * q + 1, C, sem1, carry)

    zero16 = jnp.zeros((16,), jnp.float32)
    carry = lax.fori_loop(0, npair, pair,
                          (jnp.int32(0),) + (zero16,) * NG)

    cur = carry[0]
    acc = carry[1:]
    for k in range(NG):
        acc_v[cur, pl.ds(k * 16, 16)] = acc[k]

    pltpu.sync_copy(acc_v.at[pl.ds(0, NPW)], agg_hbm.at[pl.ds(nb, NPW)])


@functools.cache
def _make_sc_agg():
    return pl.kernel(
        _sc_agg_body,
        out_type=jax.ShapeDtypeStruct((N_PAD, D), jnp.float32),
        mesh=plsc.VectorSubcoreMesh(core_axis_name="c", subcore_axis_name="s",
                                    num_cores=NC, num_subcores=NS),
        compiler_params=pltpu.CompilerParams(needs_layout_passes=False),
        scratch_types=[
            pltpu.VMEM((PTR_LEN,), jnp.int32),
            pltpu.SMEM((PTR_LEN,), jnp.int32),
            pltpu.VMEM((2 * C,), jnp.int32),
            pltpu.VMEM((2 * C, D), jnp.float32),
            pltpu.VMEM((ACC_ROWS, D), jnp.float32),
            pltpu.SemaphoreType.DMA,
            pltpu.SemaphoreType.DMA,
        ],
    )


BM = 2000


def _mm_body(x_ref, agg_ref, w_ref, o_ref):
    o_ref[...] = jnp.dot(x_ref[...] + agg_ref[...], w_ref[...],
                         preferred_element_type=jnp.float32)


def _matmul(x, agg, w):
    return pl.pallas_call(
        _mm_body,
        grid=(N // BM,),
        in_specs=[
            pl.BlockSpec((BM, D), lambda i: (i, 0)),
            pl.BlockSpec((BM, D), lambda i: (i, 0)),
            pl.BlockSpec((D, DO), lambda i: (0, 0)),
        ],
        out_specs=pl.BlockSpec((BM, DO), lambda i: (i, 0)),
        out_shape=jax.ShapeDtypeStruct((N, DO), jnp.float32),
    )(x, agg, w)


@jax.jit
def kernel(edge_ptr, src_edges, dst_nodes, input_feat, weight, neighbor_num):
    del dst_nodes, neighbor_num
    ptr_pad = jnp.pad(edge_ptr.astype(jnp.int32), (0, PTR_PAD - (N + 1)),
                      constant_values=E)
    src_pad = jnp.pad(src_edges.astype(jnp.int32), (0, SRC_PAD - E))
    feat_pad = jnp.pad(input_feat, ((0, FEAT_PAD - N), (0, 0)))
    agg = _make_sc_agg()(ptr_pad, src_pad, feat_pad)
    return _matmul(input_feat, agg, weight)


# speedup vs baseline: 8.4621x; 1.3816x over previous
import functools

import jax
import jax.numpy as jnp
from jax import lax
from jax.experimental import pallas as pl
from jax.experimental.pallas import tpu as pltpu
from jax.experimental.pallas import tpu_sc as plsc

N = 10000
E = 160000
D = 256
DO = 256
NG = D // 16

NC = 2
NS = 16
NPW = 320
NPC = NPW * NS
N_PAD = NPC * NC
C = 64
PTR_LEN = NPW + 16
PTR_PAD = N_PAD + 16
SRC_PAD = E + 2 * C
ZROW = N
ACC_ROWS = NPW + 8
FEAT_PAD = N + 8


def _sc_agg_body(ptr_hbm, src_hbm, feat_hbm, agg_hbm,
                 ptr_v, ptr_s, idx2, rows2, acc_v, sem0, sem1):
    c = lax.axis_index("c")
    s = lax.axis_index("s")
    nb = c * NPC + s * NPW

    pltpu.sync_copy(ptr_hbm.at[pl.ds(nb, PTR_LEN)], ptr_v)
    iota = lax.iota(jnp.int32, 16)
    for j in range(PTR_LEN // 16):
        v = ptr_v[pl.ds(j * 16, 16)]
        for l in range(16):
            ptr_s[j * 16 + l] = jnp.sum(jnp.where(iota == l, v, 0))

    def zrow(i, zero):
        for k in range(NG):
            acc_v[i, pl.ds(k * 16, 16)] = jnp.zeros((16,), jnp.float32)
        return zero

    lax.fori_loop(0, ACC_ROWS, zrow, jnp.int32(0))

    e0 = ptr_s[0]
    e1 = ptr_s[NPW]
    a0 = (e0 // 8) * 8
    npair = (e1 - a0 + (2 * C - 1)) // (2 * C)

    def prefetch(gc, off, sem):
        bb = a0 + gc * C
        pltpu.sync_copy(src_hbm.at[pl.ds(bb, C)], idx2.at[pl.ds(off, C)])
        for k in range(C // 16):
            sl = pl.ds(off + k * 16, 16)
            gpos = bb + k * 16 + iota
            ok = (gpos >= e0) & (gpos < e1)
            idx2[sl] = jnp.where(ok, idx2[sl], ZROW)
        pltpu.async_copy(feat_hbm.at[idx2.at[pl.ds(off, C)]],
                         rows2.at[pl.ds(off, C)], sem)

    def process(gc, off, sem, carry):
        pltpu.make_async_copy(feat_hbm.at[pl.ds(0, C)],
                              rows2.at[pl.ds(off, C)], sem).wait()
        bb = a0 + gc * C
        end = bb + C
        rp0 = off - bb

        def node_run(st):
            pos = st[0]
            cur = st[1]
            acc = st[2:]
            nxt = ptr_s[cur + 1]
            capped = cur >= NPW
            closing = (~capped) & (nxt <= end)
            stop = jnp.where(closing, nxt, end)

            def acc_edge(i, a):
                return tuple(a[k] + rows2[rp0 + i, pl.ds(k * 16, 16)]
                             for k in range(NG))

            acc = lax.fori_loop(pos, stop, acc_edge, acc)

            @pl.when(closing)
            def _():
                for k in range(NG):
                    acc_v[cur, pl.ds(k * 16, 16)] = acc[k]

            acc = tuple(jnp.where(closing, 0.0, a) for a in acc)
            cur = jnp.where(closing, cur + 1, cur)
            return (stop, cur) + acc

        st = lax.while_loop(lambda st: st[0] < end, node_run,
                            (bb, carry[0]) + tuple(carry[1:]))
        return st[1:]

    @pl.when(npair > 0)
    def _():
        prefetch(0, 0, sem0)

    def pair(q, carry):
        prefetch(2 * q + 1, C, sem1)
        carry = process(2 * q, 0, sem0, carry)

        @pl.when(2 * q + 2 < 2 * npair)
        def _():
            prefetch(2 * q + 2, 0, sem0)

        return process(2 * q + 1, C, sem1, carry)

    zero16 = jnp.zeros((16,), jnp.float32)
    carry = lax.fori_loop(0, npair, pair,
                          (jnp.int32(0),) + (zero16,) * NG)

    cur = carry[0]
    acc = carry[1:]
    for k in range(NG):
        acc_v[cur, pl.ds(k * 16, 16)] = acc[k]

    pltpu.sync_copy(acc_v.at[pl.ds(0, NPW)], agg_hbm.at[pl.ds(nb, NPW)])


@functools.cache
def _make_sc_agg():
    return pl.kernel(
        _sc_agg_body,
        out_type=jax.ShapeDtypeStruct((N_PAD, D), jnp.float32),
        mesh=plsc.VectorSubcoreMesh(core_axis_name="c", subcore_axis_name="s",
                                    num_cores=NC, num_subcores=NS),
        compiler_params=pltpu.CompilerParams(needs_layout_passes=False),
        scratch_types=[
            pltpu.VMEM((PTR_LEN,), jnp.int32),
            pltpu.SMEM((PTR_LEN,), jnp.int32),
            pltpu.VMEM((2 * C,), jnp.int32),
            pltpu.VMEM((2 * C, D), jnp.float32),
            pltpu.VMEM((ACC_ROWS, D), jnp.float32),
            pltpu.SemaphoreType.DMA,
            pltpu.SemaphoreType.DMA,
        ],
    )


BM = 2000


def _mm_body(x_ref, agg_ref, w_ref, o_ref):
    o_ref[...] = jnp.dot(x_ref[...] + agg_ref[...], w_ref[...],
                         preferred_element_type=jnp.float32)


def _matmul(x, agg, w):
    return pl.pallas_call(
        _mm_body,
        grid=(N // BM,),
        in_specs=[
            pl.BlockSpec((BM, D), lambda i: (i, 0)),
            pl.BlockSpec((BM, D), lambda i: (i, 0)),
            pl.BlockSpec((D, DO), lambda i: (0, 0)),
        ],
        out_specs=pl.BlockSpec((BM, DO), lambda i: (i, 0)),
        out_shape=jax.ShapeDtypeStruct((N, DO), jnp.float32),
    )(x, agg, w)


@jax.jit
def kernel(edge_ptr, src_edges, dst_nodes, input_feat, weight, neighbor_num):
    del dst_nodes, neighbor_num
    ptr_pad = jnp.pad(edge_ptr.astype(jnp.int32), (0, PTR_PAD - (N + 1)),
                      constant_values=E)
    src_pad = jnp.pad(src_edges.astype(jnp.int32), (0, SRC_PAD - E))
    feat_pad = jnp.pad(input_feat, ((0, FEAT_PAD - N), (0, 0)))
    agg = _make_sc_agg()(ptr_pad, src_pad, feat_pad)
    return _matmul(input_feat, agg, weight)
